# Initial kernel scaffold; baseline (speedup 1.0000x reference)
import jax
import jax.numpy as jnp
from jax.experimental import pallas as pl


def kernel(observation, tables):
    raise NotImplementedError("write your pallas kernel here")


# speedup vs baseline: 1.2135x; 1.2135x over previous
import functools

import jax
import jax.numpy as jnp
from jax import lax
from jax.experimental import pallas as pl
from jax.experimental.pallas import tpu as pltpu
from jax.experimental.pallas import tpu_sc as plsc

N_FIELDS = 26
VOCAB = 100000
DIM = 32
BATCH = 16384

NC = 2
NS = 16
NW = NC * NS
TOTAL = BATCH * N_FIELDS
ROWS_PER_W = TOTAL // NW
CHUNK = 128
K = 8
GROUP = K * CHUNK
NCHUNK = ROWS_PER_W // CHUNK
NGROUP = ROWS_PER_W // GROUP


def _sc_gather(gidx, table_flat):
    mesh = plsc.VectorSubcoreMesh(core_axis_name="c", subcore_axis_name="s")

    @functools.partial(
        pl.kernel,
        out_type=jax.ShapeDtypeStruct((NW, NGROUP, K, CHUNK, DIM), jnp.float32),
        mesh=mesh,
        scratch_types=[
            pltpu.VMEM((NCHUNK, CHUNK), jnp.int32),
            pltpu.VMEM((2, K, CHUNK, DIM), jnp.float32),
            pltpu.SemaphoreType.DMA,
            pltpu.SemaphoreType.DMA,
        ],
        compiler_params=pltpu.CompilerParams(use_tc_tiling_on_sc=False),
    )
    def k(gidx_hbm, table_hbm, out_hbm, idx_v, buf, sem0, sem1):
        wid = lax.axis_index("s") * NC + lax.axis_index("c")
        sems = (sem0, sem1)
        pltpu.sync_copy(gidx_hbm.at[wid], idx_v)

        def fire(g):
            b = g % 2
            return [
                pltpu.async_copy(
                    table_hbm.at[idx_v.at[g * K + kk]], buf.at[b, kk], sems[b])
                for kk in range(K)
            ]

        handles = fire(0)
        for g in range(NGROUP):
            nxt = fire(g + 1) if g + 1 < NGROUP else []
            for h in handles:
                h.wait()
            pltpu.sync_copy(buf.at[g % 2], out_hbm.at[wid, g])
            handles = nxt

    return k(gidx, table_flat)


def kernel(observation, tables):
    offsets = (jnp.arange(N_FIELDS, dtype=jnp.int32) * VOCAB)[None, :]
    gidx = (observation + offsets).reshape(NW, NCHUNK, CHUNK)
    table_flat = tables.reshape(N_FIELDS * VOCAB, DIM)
    out = _sc_gather(gidx, table_flat)
    return out.reshape(BATCH, N_FIELDS * DIM)


# speedup vs baseline: 1.7628x; 1.4527x over previous
import functools

import jax
import jax.numpy as jnp
from jax import lax
from jax.experimental import pallas as pl
from jax.experimental.pallas import tpu as pltpu
from jax.experimental.pallas import tpu_sc as plsc

N_FIELDS = 26
VOCAB = 100000
DIM = 32
BATCH = 16384

NC = 2
NS = 16
NW = NC * NS
TOTAL = BATCH * N_FIELDS
ROWS_PER_W = TOTAL // NW
CHUNK = 128
K = 8
GROUP = K * CHUNK
NCHUNK = ROWS_PER_W // CHUNK
NGROUP = ROWS_PER_W // GROUP


Q = VOCAB // 4


def _transpose_body(x_ref, y_ref):
    j = pl.program_id(1)
    s = Q // 4
    for jj in range(4):
        @pl.when(j == jj)
        def _():
            for k in range(4):
                xj = x_ref[0, :, jj * Q + k * s:jj * Q + (k + 1) * s]
                y_ref[k * s:(k + 1) * s, jj * DIM:(jj + 1) * DIM] = xj.T


def _tc_transpose(tab_t):
    return pl.pallas_call(
        _transpose_body,
        grid=(N_FIELDS, 4),
        in_specs=[pl.BlockSpec((1, DIM, VOCAB), lambda f, j: (f, 0, 0))],
        out_specs=pl.BlockSpec((VOCAB * DIM // 128, 128),
                               lambda f, j: (f, 0)),
        out_shape=jax.ShapeDtypeStruct((N_FIELDS * VOCAB * DIM // 128, 128),
                                       jnp.float32),
    )(tab_t)


def _sc_gather(gidx, table_flat):
    mesh = plsc.VectorSubcoreMesh(core_axis_name="c", subcore_axis_name="s")

    @functools.partial(
        pl.kernel,
        out_type=jax.ShapeDtypeStruct((NW, NGROUP, K, CHUNK, DIM), jnp.float32),
        mesh=mesh,
        scratch_types=[
            pltpu.VMEM((NCHUNK, CHUNK), jnp.int32),
            pltpu.VMEM((2, K, CHUNK, DIM), jnp.float32),
            pltpu.SemaphoreType.DMA,
            pltpu.SemaphoreType.DMA,
        ],
        compiler_params=pltpu.CompilerParams(use_tc_tiling_on_sc=False),
    )
    def k(gidx_hbm, table_hbm, out_hbm, idx_v, buf, sem0, sem1):
        wid = lax.axis_index("s") * NC + lax.axis_index("c")
        sems = (sem0, sem1)
        pltpu.sync_copy(gidx_hbm.at[wid], idx_v)

        def fire(g):
            b = g % 2
            return [
                pltpu.async_copy(
                    table_hbm.at[idx_v.at[g * K + kk]], buf.at[b, kk], sems[b])
                for kk in range(K)
            ]

        handles = fire(0)
        for g in range(NGROUP):
            nxt = fire(g + 1) if g + 1 < NGROUP else []
            for h in handles:
                h.wait()
            pltpu.sync_copy(buf.at[g % 2], out_hbm.at[wid, g])
            handles = nxt

    return k(gidx, table_flat)


def kernel(observation, tables):
    offsets = (jnp.arange(N_FIELDS, dtype=jnp.int32) * VOCAB)[None, :]
    gidx = (offsets + (observation % Q) * 4 + observation // Q
            ).reshape(NW, NCHUNK, CHUNK)
    tab_t = tables.transpose(0, 2, 1)
    table_flat = _tc_transpose(tab_t).reshape(N_FIELDS * VOCAB, DIM)
    out = _sc_gather(gidx, table_flat)
    return out.reshape(BATCH, N_FIELDS * DIM)
